# baseline (speedup 1.0000x reference)
import functools

import jax
import jax.numpy as jnp
from jax import lax
from jax.experimental import pallas as pl
from jax.experimental.pallas import tpu as pltpu
from jax.experimental.pallas import tpu_sc as plsc

B, H, S, D, T = 8, 16, 2048, 128, 16

NC, NS = 2, 16
NW = NC * NS
ROWS = B * H * T
RPW = ROWS // NW
GPW = RPW // T

_mesh = plsc.VectorSubcoreMesh(core_axis_name="c", subcore_axis_name="s")


@functools.partial(
    pl.kernel,
    out_type=(),
    mesh=_mesh,
    scratch_types=[
        pltpu.VMEM((T,), jnp.int32),
        pltpu.VMEM((RPW,), jnp.int32),
        pltpu.VMEM((RPW, D), jnp.float32),
        pltpu.SemaphoreType.DMA,
    ],
)
def _scatter_one(c_ref, n_hbm, pos_hbm, pos_v, idx_v, rows_v, sem):
    wid = lax.axis_index("s") * NC + lax.axis_index("c")
    base = wid * RPW
    cp_rows = pltpu.async_copy(n_hbm.at[pl.ds(base, RPW)], rows_v, sem)
    pltpu.sync_copy(pos_hbm, pos_v)
    pos = pos_v[...]
    for g in range(GPW):
        bh = wid * GPW + g
        idx_v[pl.ds(g * T, T)] = pos + bh * S
    cp_rows.wait()
    pltpu.async_copy(rows_v, c_ref.at[idx_v], sem).wait()


def kernel(cache_k, cache_v, new_k, new_v, position_ids):
    pos = position_ids.astype(jnp.int32)
    ck = jax.new_ref(cache_k.reshape(B * H * S, D))
    _scatter_one(ck, new_k.reshape(ROWS, D), pos)
    cv = jax.new_ref(cache_v.reshape(B * H * S, D))
    _scatter_one(cv, new_v.reshape(ROWS, D), pos)
    return (ck[...].reshape(B, H, S, D), cv[...].reshape(B, H, S, D))
